# Initial kernel scaffold; baseline (speedup 1.0000x reference)
import jax
import jax.numpy as jnp
from jax.experimental import pallas as pl


def kernel(x, edge_index, W_l1, b_l1, W_r1, W_l2, b_l2, W_r2, ln1_g, ln1_b, ln2_g, ln2_b, W_c1, b_c1, W_c2, b_c2):
    raise NotImplementedError("write your pallas kernel here")


# speedup vs baseline: 3.0430x; 3.0430x over previous
import functools

import jax
import jax.numpy as jnp
from jax import lax
from jax.experimental import pallas as pl
from jax.experimental.pallas import tpu as pltpu
from jax.experimental.pallas import tpu_sc as plsc

N_NODES = 10000
N_EDGES = 320000
F_IN = 128
HID = 256

NC = 2
NS = 16
CHUNK = 128

N_PAD = 10112
E_PAD = 327680
G = 8
EV_T1 = E_PAD // (NC * NS)
EV_T2 = E_PAD // NS
NCH1 = EV_T1 // CHUNK
NCH2 = EV_T2 // CHUNK
ROWS_T = N_PAD // NS


def _fill_zeros(ref, rows):
    z = jnp.zeros((16,), jnp.float32)
    for i in range(rows):
        for j in range(ref.shape[-1] // 16):
            ref[i, pl.ds(j * 16, 16)] = z


def _edge_loop(nch, wid, src_hbm, dst_hbm, ev_base, table_hbm, src_buf,
               dst_buf, rows, acc_sh, deg_sh, ones):
    def body(g, carry):
        pltpu.sync_copy(src_hbm.at[pl.ds(ev_base + g * (G * CHUNK),
                                         G * CHUNK)], src_buf)
        pltpu.sync_copy(dst_hbm.at[wid, pl.ds(g * G, G)], dst_buf)
        for j in range(G):
            pltpu.sync_copy(
                table_hbm.at[src_buf.at[pl.ds(j * CHUNK, CHUNK)]], rows)
            pltpu.sync_copy(rows, acc_sh.at[dst_buf.at[j]], add=True)
            if deg_sh is not None:
                pltpu.sync_copy(ones, deg_sh.at[dst_buf.at[j]], add=True)
        return carry

    lax.fori_loop(0, nch // G, body, 0)


def _zero_acc(tid, acc_sh, deg_sh, zrow):
    def zbody(k, carry):
        r0 = tid * ROWS_T + k * 8
        pltpu.sync_copy(zrow, acc_sh.at[pl.ds(r0, 8)])
        if deg_sh is not None:
            pltpu.sync_copy(zrow.at[0, pl.ds(0, 8)], deg_sh.at[pl.ds(r0, 8)])
        return carry

    lax.fori_loop(0, ROWS_T // 8, zbody, 0)


def _writeback(tid, acc_sh, out_hbm, core, rows):
    r0 = tid * ROWS_T
    for k in range(4):
        pltpu.sync_copy(acc_sh.at[pl.ds(r0 + k * 128, 128)], rows)
        pltpu.sync_copy(rows, out_hbm.at[core, pl.ds(r0 + k * 128, 128)])
    tail = rows.at[pl.ds(0, ROWS_T - 512)]
    pltpu.sync_copy(acc_sh.at[pl.ds(r0 + 512, ROWS_T - 512)], tail)
    pltpu.sync_copy(tail, out_hbm.at[core, pl.ds(r0 + 512, ROWS_T - 512)])


@functools.partial(
    pl.kernel,
    mesh=plsc.VectorSubcoreMesh(core_axis_name="c", subcore_axis_name="s"),
    out_type=(
        jax.ShapeDtypeStruct((NC, N_PAD, F_IN), jnp.float32),
        jax.ShapeDtypeStruct((NC * N_PAD,), jnp.float32),
    ),
    scratch_types=(
        pltpu.VMEM_SHARED((N_PAD, F_IN), jnp.float32),
        pltpu.VMEM_SHARED((N_PAD,), jnp.float32),
        pltpu.VMEM((G * CHUNK,), jnp.int32),
        pltpu.VMEM((G, CHUNK), jnp.int32),
        pltpu.VMEM((CHUNK, F_IN), jnp.float32),
        pltpu.VMEM((8, F_IN), jnp.float32),
        pltpu.VMEM((CHUNK,), jnp.float32),
        pltpu.VMEM((ROWS_T,), jnp.float32),
    ),
)
def _sc_agg1(x_hbm, src_hbm, dst_hbm, agg_out, deg_out,
             acc_sh, deg_sh, src_buf, dst_buf, rows, zrow, ones, degb):
    c = lax.axis_index("c")
    s = lax.axis_index("s")
    tid = s
    wid = c * NS + s

    _fill_zeros(zrow, 8)
    o = jnp.full((16,), 1.0, jnp.float32)
    for j in range(CHUNK // 16):
        ones[pl.ds(j * 16, 16)] = o

    _zero_acc(tid, acc_sh, deg_sh, zrow)
    plsc.subcore_barrier()

    _edge_loop(NCH1, wid, src_hbm, dst_hbm, wid * EV_T1, x_hbm, src_buf,
               dst_buf, rows, acc_sh, deg_sh, ones)
    plsc.subcore_barrier()

    _writeback(tid, acc_sh, agg_out, c, rows)
    pltpu.sync_copy(deg_sh.at[pl.ds(tid * ROWS_T, ROWS_T)], degb)
    pltpu.sync_copy(degb, deg_out.at[pl.ds(c * N_PAD + tid * ROWS_T, ROWS_T)])


@functools.partial(
    pl.kernel,
    mesh=plsc.VectorSubcoreMesh(core_axis_name="c", subcore_axis_name="s"),
    out_type=jax.ShapeDtypeStruct((NC, N_PAD, HID // 2), jnp.float32),
    scratch_types=(
        pltpu.VMEM_SHARED((N_PAD, HID // 2), jnp.float32),
        pltpu.VMEM((G * CHUNK,), jnp.int32),
        pltpu.VMEM((G, CHUNK), jnp.int32),
        pltpu.VMEM((CHUNK, HID // 2), jnp.float32),
        pltpu.VMEM((8, HID // 2), jnp.float32),
    ),
)
def _sc_agg2(h1a_hbm, h1b_hbm, src_hbm, dst_hbm, agg_out,
             acc_sh, src_buf, dst_buf, rows, zrow):
    c = lax.axis_index("c")
    s = lax.axis_index("s")
    tid = s

    _fill_zeros(zrow, 8)
    _zero_acc(tid, acc_sh, None, zrow)
    plsc.subcore_barrier()

    @pl.when(c == 0)
    def _():
        _edge_loop(NCH2, s, src_hbm, dst_hbm, s * EV_T2, h1a_hbm, src_buf,
                   dst_buf, rows, acc_sh, None, None)

    @pl.when(c == 1)
    def _():
        _edge_loop(NCH2, s, src_hbm, dst_hbm, s * EV_T2, h1b_hbm, src_buf,
                   dst_buf, rows, acc_sh, None, None)

    plsc.subcore_barrier()
    _writeback(tid, acc_sh, agg_out, c, rows)


_BLK = 1264
_GRID = N_PAD // _BLK


def _tc1_body(agg_ref, degp_ref, x_ref, wl_ref, wr_ref, b_ref, g_ref,
              bb_ref, h1a_ref, h1b_ref, deg_ref):
    deg = jnp.maximum(degp_ref[0] + degp_ref[1], 1.0)
    mean = (agg_ref[0] + agg_ref[1]) / deg
    h = (jnp.dot(mean, wl_ref[...], preferred_element_type=jnp.float32)
         + jnp.dot(x_ref[...], wr_ref[...], preferred_element_type=jnp.float32)
         + b_ref[...])
    mu = jnp.mean(h, axis=1, keepdims=True)
    var = jnp.mean((h - mu) * (h - mu), axis=1, keepdims=True)
    h = (h - mu) * lax.rsqrt(var + 1e-5) * g_ref[...] + bb_ref[...]
    h = jnp.maximum(h, 0.0)
    h1a_ref[...] = h[:, :HID // 2]
    h1b_ref[...] = h[:, HID // 2:]
    deg_ref[...] = deg


def _tc_layer1(agg, degp, x, wl, wr, b, g, bb):
    full = lambda s: pl.BlockSpec(s, lambda i: (0,) * len(s))
    return pl.pallas_call(
        _tc1_body,
        grid=(_GRID,),
        in_specs=[
            pl.BlockSpec((NC, _BLK, F_IN), lambda i: (0, i, 0)),
            pl.BlockSpec((NC, _BLK, 1), lambda i: (0, i, 0)),
            pl.BlockSpec((_BLK, F_IN), lambda i: (i, 0)),
            full((F_IN, HID)), full((F_IN, HID)), full((1, HID)),
            full((1, HID)), full((1, HID)),
        ],
        out_specs=[
            pl.BlockSpec((_BLK, HID // 2), lambda i: (i, 0)),
            pl.BlockSpec((_BLK, HID // 2), lambda i: (i, 0)),
            pl.BlockSpec((_BLK, 1), lambda i: (i, 0)),
        ],
        out_shape=[
            jax.ShapeDtypeStruct((N_PAD, HID // 2), jnp.float32),
            jax.ShapeDtypeStruct((N_PAD, HID // 2), jnp.float32),
            jax.ShapeDtypeStruct((N_PAD, 1), jnp.float32),
        ],
    )(agg, degp, x, wl, wr, b, g, bb)


def _tc2_body(h1a_ref, h1b_ref, agg_ref, deg_ref, wl_ref, wr_ref, b_ref,
              g_ref, bb_ref, wc1_ref, bc1_ref, wc2_ref, bc2_ref, out_ref):
    rdeg = 1.0 / deg_ref[...]
    wl = wl_ref[...]
    wr = wr_ref[...]
    dot = lambda a, w: jnp.dot(a, w, preferred_element_type=jnp.float32)
    h = (dot(agg_ref[0] * rdeg, wl[:HID // 2])
         + dot(agg_ref[1] * rdeg, wl[HID // 2:])
         + dot(h1a_ref[...], wr[:HID // 2])
         + dot(h1b_ref[...], wr[HID // 2:])
         + b_ref[...])
    mu = jnp.mean(h, axis=1, keepdims=True)
    var = jnp.mean((h - mu) * (h - mu), axis=1, keepdims=True)
    h = (h - mu) * lax.rsqrt(var + 1e-5) * g_ref[...] + bb_ref[...]
    h = jnp.maximum(h, 0.0)
    z = jnp.maximum(dot(h, wc1_ref[...]) + bc1_ref[...], 0.0)
    out_ref[...] = dot(z, wc2_ref[...]) + bc2_ref[...]


def _tc_layer2(h1a, h1b, agg, deg, wl, wr, b, g, bb, wc1, bc1, wc2, bc2):
    full = lambda s: pl.BlockSpec(s, lambda i: (0,) * len(s))
    return pl.pallas_call(
        _tc2_body,
        grid=(_GRID,),
        in_specs=[
            pl.BlockSpec((_BLK, HID // 2), lambda i: (i, 0)),
            pl.BlockSpec((_BLK, HID // 2), lambda i: (i, 0)),
            pl.BlockSpec((NC, _BLK, HID // 2), lambda i: (0, i, 0)),
            pl.BlockSpec((_BLK, 1), lambda i: (i, 0)),
            full((HID, HID)), full((HID, HID)), full((1, HID)),
            full((1, HID)), full((1, HID)),
            full((HID, HID // 2)), full((1, HID // 2)),
            full((HID // 2, HID // 2)), full((1, HID // 2)),
        ],
        out_specs=pl.BlockSpec((_BLK, HID // 2), lambda i: (i, 0)),
        out_shape=jax.ShapeDtypeStruct((N_PAD, HID // 2), jnp.float32),
    )(h1a, h1b, agg, deg, wl, wr, b, g, bb, wc1, bc1, wc2, bc2)


def kernel(x, edge_index, W_l1, b_l1, W_r1, W_l2, b_l2, W_r2,
           ln1_g, ln1_b, ln2_g, ln2_b, W_c1, b_c1, W_c2, b_c2):
    src = edge_index[0]
    dst = edge_index[1]
    pad_e = E_PAD - N_EDGES
    src_p = jnp.concatenate([src, jnp.zeros((pad_e,), jnp.int32)])
    dst_p = jnp.concatenate(
        [dst, jnp.full((pad_e,), N_NODES, jnp.int32)])
    dst3d_1 = dst_p.reshape(NC * NS, NCH1, CHUNK)
    dst3d_2 = dst_p.reshape(NS, NCH2, CHUNK)
    x_p = jnp.pad(x, ((0, N_PAD - N_NODES), (0, 0)))

    agg1, degp = _sc_agg1(x_p, src_p, dst3d_1)

    row = lambda v: v.reshape(1, -1)
    h1a, h1b, deg = _tc_layer1(
        agg1, degp.reshape(NC, N_PAD, 1), x_p, W_l1, W_r1, row(b_l1),
        row(ln1_g), row(ln1_b))

    agg2 = _sc_agg2(h1a, h1b, src_p, dst3d_2)

    wc2_p = jnp.pad(W_c2, ((0, 0), (0, HID // 2 - W_c2.shape[1])))
    bc2_p = row(jnp.pad(b_c2, (0, HID // 2 - b_c2.shape[0])))
    logits_p = _tc_layer2(h1a, h1b, agg2, deg, W_l2, W_r2, row(b_l2),
                          row(ln2_g), row(ln2_b), W_c1, row(b_c1),
                          wc2_p, bc2_p)
    return logits_p[:N_NODES, :W_c2.shape[1]]


# speedup vs baseline: 3.4448x; 1.1320x over previous
import functools

import jax
import jax.numpy as jnp
from jax import lax
from jax.experimental import pallas as pl
from jax.experimental.pallas import tpu as pltpu
from jax.experimental.pallas import tpu_sc as plsc

N_NODES = 10000
N_EDGES = 320000
F_IN = 128
HID = 256

NC = 2
NS = 16
CHUNK = 128

N_PAD = 10112
E_PAD = 327680
G = 8
EV_T1 = E_PAD // (NC * NS)
EV_T2 = E_PAD // NS
NCH1 = EV_T1 // CHUNK
NCH2 = EV_T2 // CHUNK
ROWS_T = N_PAD // NS


def _fill_zeros(ref, rows):
    z = jnp.zeros((16,), jnp.float32)
    for i in range(rows):
        for j in range(ref.shape[-1] // 16):
            ref[i, pl.ds(j * 16, 16)] = z


def _edge_pipeline(nch, wid, src_hbm, dst_hbm, ev_base, table_hbm, src_b,
                   dst_b, rows, acc_sh, deg_sh, ones, ssem, dsem, rsem,
                   idx_off):
    GC = G * CHUNK
    ngroups = nch // G

    def idx_load(g, b):
        s_d = pltpu.async_copy(src_hbm.at[pl.ds(ev_base + g * GC, GC)],
                               src_b.at[b], ssem[b])
        d_d = pltpu.async_copy(dst_hbm.at[wid, pl.ds(g * G, G)],
                               dst_b.at[b], dsem[b])
        return s_d, d_d

    def wait_idx(b):
        pltpu.make_async_copy(src_hbm.at[pl.ds(0, GC)], src_b.at[b],
                              ssem[b]).wait()
        pltpu.make_async_copy(dst_hbm.at[0, pl.ds(0, G)], dst_b.at[b],
                              dsem[b]).wait()

    def offset_idx(b):
        if idx_off is not None:
            off = jnp.zeros((16,), jnp.int32) + idx_off
            for k in range(GC // 16):
                src_b[b, pl.ds(k * 16, 16)] = src_b[b, pl.ds(k * 16, 16)] + off

    def issue_gather(idx_slice, p):
        pltpu.async_copy(table_hbm.at[idx_slice], rows.at[p], rsem[p])

    def wait_gather(p):
        pltpu.make_async_copy(table_hbm.at[src_b.at[0, pl.ds(0, CHUNK)]],
                              rows.at[p], rsem[p]).wait()

    s_d, d_d = idx_load(0, 0)
    s_d.wait()
    d_d.wait()
    offset_idx(0)
    idx_load(1, 1)
    issue_gather(src_b.at[0, pl.ds(0, CHUNK)], 0)

    def body(bi, carry):
        for half in range(2):
            gbase = bi * 2 + half

            for j in range(G):
                p = j % 2
                ch = gbase * G + j
                wait_gather(p)
                if j < G - 1:
                    nslice = src_b.at[half, pl.ds((j + 1) * CHUNK, CHUNK)]

                    @pl.when(ch + 1 < nch)
                    def _():
                        issue_gather(nslice, 1 - p)
                else:
                    nslice = src_b.at[1 - half, pl.ds(0, CHUNK)]

                    @pl.when(ch + 1 < nch)
                    def _():
                        wait_idx(1 - half)
                        offset_idx(1 - half)
                        issue_gather(nslice, 1 - p)

                pltpu.sync_copy(rows.at[p], acc_sh.at[dst_b.at[half, j]],
                                add=True)
                if deg_sh is not None:
                    pltpu.sync_copy(ones, deg_sh.at[dst_b.at[half, j]],
                                    add=True)

            @pl.when(gbase + 2 < ngroups)
            def _():
                idx_load(gbase + 2, half)

        return carry

    lax.fori_loop(0, ngroups // 2, body, 0)


def _zero_acc(tid, acc_sh, deg_sh, zrow):
    def zbody(k, carry):
        r0 = tid * ROWS_T + k * 8
        pltpu.sync_copy(zrow, acc_sh.at[pl.ds(r0, 8)])
        if deg_sh is not None:
            pltpu.sync_copy(zrow.at[0, pl.ds(0, 8)], deg_sh.at[pl.ds(r0, 8)])
        return carry

    lax.fori_loop(0, ROWS_T // 8, zbody, 0)


def _writeback(tid, acc_sh, out_hbm, core, rows):
    r0 = tid * ROWS_T
    for k in range(4):
        pltpu.sync_copy(acc_sh.at[pl.ds(r0 + k * 128, 128)], rows)
        pltpu.sync_copy(rows, out_hbm.at[core, pl.ds(r0 + k * 128, 128)])
    tail = rows.at[pl.ds(0, ROWS_T - 512)]
    pltpu.sync_copy(acc_sh.at[pl.ds(r0 + 512, ROWS_T - 512)], tail)
    pltpu.sync_copy(tail, out_hbm.at[core, pl.ds(r0 + 512, ROWS_T - 512)])


@functools.partial(
    pl.kernel,
    mesh=plsc.VectorSubcoreMesh(core_axis_name="c", subcore_axis_name="s"),
    out_type=(
        jax.ShapeDtypeStruct((NC, N_PAD, F_IN), jnp.float32),
        jax.ShapeDtypeStruct((NC * N_PAD,), jnp.float32),
    ),
    scratch_types=(
        pltpu.VMEM_SHARED((N_PAD, F_IN), jnp.float32),
        pltpu.VMEM_SHARED((N_PAD,), jnp.float32),
        pltpu.VMEM((2, G * CHUNK), jnp.int32),
        pltpu.VMEM((2, G, CHUNK), jnp.int32),
        pltpu.VMEM((2, CHUNK, F_IN), jnp.float32),
        pltpu.VMEM((8, F_IN), jnp.float32),
        pltpu.VMEM((CHUNK,), jnp.float32),
        pltpu.VMEM((ROWS_T,), jnp.float32),
        pltpu.SemaphoreType.DMA,
        pltpu.SemaphoreType.DMA,
        pltpu.SemaphoreType.DMA,
        pltpu.SemaphoreType.DMA,
        pltpu.SemaphoreType.DMA,
        pltpu.SemaphoreType.DMA,
    ),
)
def _sc_agg1(x_hbm, src_hbm, dst_hbm, agg_out, deg_out,
             acc_sh, deg_sh, src_b, dst_b, rows, zrow, ones, degb,
             ssem0, ssem1, dsem0, dsem1, rsem0, rsem1):
    c = lax.axis_index("c")
    s = lax.axis_index("s")
    tid = s
    wid = c * NS + s

    _fill_zeros(zrow, 8)
    o = jnp.full((16,), 1.0, jnp.float32)
    for j in range(CHUNK // 16):
        ones[pl.ds(j * 16, 16)] = o

    _zero_acc(tid, acc_sh, deg_sh, zrow)
    plsc.subcore_barrier()

    _edge_pipeline(NCH1, wid, src_hbm, dst_hbm, wid * EV_T1, x_hbm, src_b,
                   dst_b, rows, acc_sh, deg_sh, ones, (ssem0, ssem1),
                   (dsem0, dsem1), (rsem0, rsem1), None)
    plsc.subcore_barrier()

    _writeback(tid, acc_sh, agg_out, c, rows.at[0])
    pltpu.sync_copy(deg_sh.at[pl.ds(tid * ROWS_T, ROWS_T)], degb)
    pltpu.sync_copy(degb, deg_out.at[pl.ds(c * N_PAD + tid * ROWS_T, ROWS_T)])


@functools.partial(
    pl.kernel,
    mesh=plsc.VectorSubcoreMesh(core_axis_name="c", subcore_axis_name="s"),
    out_type=jax.ShapeDtypeStruct((NC, N_PAD, HID // 2), jnp.float32),
    scratch_types=(
        pltpu.VMEM_SHARED((N_PAD, HID // 2), jnp.float32),
        pltpu.VMEM((2, G * CHUNK), jnp.int32),
        pltpu.VMEM((2, G, CHUNK), jnp.int32),
        pltpu.VMEM((2, CHUNK, HID // 2), jnp.float32),
        pltpu.VMEM((8, HID // 2), jnp.float32),
        pltpu.SemaphoreType.DMA,
        pltpu.SemaphoreType.DMA,
        pltpu.SemaphoreType.DMA,
        pltpu.SemaphoreType.DMA,
        pltpu.SemaphoreType.DMA,
        pltpu.SemaphoreType.DMA,
    ),
)
def _sc_agg2(h1_hbm, src_hbm, dst_hbm, agg_out,
             acc_sh, src_b, dst_b, rows, zrow,
             ssem0, ssem1, dsem0, dsem1, rsem0, rsem1):
    c = lax.axis_index("c")
    s = lax.axis_index("s")
    tid = s

    _fill_zeros(zrow, 8)
    _zero_acc(tid, acc_sh, None, zrow)
    plsc.subcore_barrier()

    _edge_pipeline(NCH2, s, src_hbm, dst_hbm, s * EV_T2, h1_hbm, src_b,
                   dst_b, rows, acc_sh, None, None, (ssem0, ssem1),
                   (dsem0, dsem1), (rsem0, rsem1), c * N_PAD)
    plsc.subcore_barrier()

    _writeback(tid, acc_sh, agg_out, c, rows.at[0])


_BLK = 1264
_GRID = N_PAD // _BLK


def _tc1_body(agg_ref, degp_ref, x_ref, wl_ref, wr_ref, b_ref, g_ref,
              bb_ref, h1_ref, deg_ref):
    deg = jnp.maximum(degp_ref[0] + degp_ref[1], 1.0)
    mean = (agg_ref[0] + agg_ref[1]) / deg
    h = (jnp.dot(mean, wl_ref[...], preferred_element_type=jnp.float32)
         + jnp.dot(x_ref[...], wr_ref[...], preferred_element_type=jnp.float32)
         + b_ref[...])
    mu = jnp.mean(h, axis=1, keepdims=True)
    var = jnp.mean((h - mu) * (h - mu), axis=1, keepdims=True)
    h = (h - mu) * lax.rsqrt(var + 1e-5) * g_ref[...] + bb_ref[...]
    h = jnp.maximum(h, 0.0)
    h1_ref[0] = h[:, :HID // 2]
    h1_ref[1] = h[:, HID // 2:]
    deg_ref[...] = deg


def _tc_layer1(agg, degp, x, wl, wr, b, g, bb):
    full = lambda s: pl.BlockSpec(s, lambda i: (0,) * len(s))
    return pl.pallas_call(
        _tc1_body,
        grid=(_GRID,),
        in_specs=[
            pl.BlockSpec((NC, _BLK, F_IN), lambda i: (0, i, 0)),
            pl.BlockSpec((NC, _BLK, 1), lambda i: (0, i, 0)),
            pl.BlockSpec((_BLK, F_IN), lambda i: (i, 0)),
            full((F_IN, HID)), full((F_IN, HID)), full((1, HID)),
            full((1, HID)), full((1, HID)),
        ],
        out_specs=[
            pl.BlockSpec((NC, _BLK, HID // 2), lambda i: (0, i, 0)),
            pl.BlockSpec((_BLK, 1), lambda i: (i, 0)),
        ],
        out_shape=[
            jax.ShapeDtypeStruct((NC, N_PAD, HID // 2), jnp.float32),
            jax.ShapeDtypeStruct((N_PAD, 1), jnp.float32),
        ],
    )(agg, degp, x, wl, wr, b, g, bb)


def _tc2_body(h1_ref, agg_ref, deg_ref, wl_ref, wr_ref, b_ref,
              g_ref, bb_ref, wc1_ref, bc1_ref, wc2_ref, bc2_ref, out_ref):
    rdeg = 1.0 / deg_ref[...]
    wl = wl_ref[...]
    wr = wr_ref[...]
    dot = lambda a, w: jnp.dot(a, w, preferred_element_type=jnp.float32)
    h = (dot(agg_ref[0] * rdeg, wl[:HID // 2])
         + dot(agg_ref[1] * rdeg, wl[HID // 2:])
         + dot(h1_ref[0], wr[:HID // 2])
         + dot(h1_ref[1], wr[HID // 2:])
         + b_ref[...])
    mu = jnp.mean(h, axis=1, keepdims=True)
    var = jnp.mean((h - mu) * (h - mu), axis=1, keepdims=True)
    h = (h - mu) * lax.rsqrt(var + 1e-5) * g_ref[...] + bb_ref[...]
    h = jnp.maximum(h, 0.0)
    z = jnp.maximum(dot(h, wc1_ref[...]) + bc1_ref[...], 0.0)
    out_ref[...] = dot(z, wc2_ref[...]) + bc2_ref[...]


def _tc_layer2(h1, agg, deg, wl, wr, b, g, bb, wc1, bc1, wc2, bc2):
    full = lambda s: pl.BlockSpec(s, lambda i: (0,) * len(s))
    return pl.pallas_call(
        _tc2_body,
        grid=(_GRID,),
        in_specs=[
            pl.BlockSpec((NC, _BLK, HID // 2), lambda i: (0, i, 0)),
            pl.BlockSpec((NC, _BLK, HID // 2), lambda i: (0, i, 0)),
            pl.BlockSpec((_BLK, 1), lambda i: (i, 0)),
            full((HID, HID)), full((HID, HID)), full((1, HID)),
            full((1, HID)), full((1, HID)),
            full((HID, HID // 2)), full((1, HID // 2)),
            full((HID // 2, HID // 2)), full((1, HID // 2)),
        ],
        out_specs=pl.BlockSpec((_BLK, HID // 2), lambda i: (i, 0)),
        out_shape=jax.ShapeDtypeStruct((N_PAD, HID // 2), jnp.float32),
    )(h1, agg, deg, wl, wr, b, g, bb, wc1, bc1, wc2, bc2)


def kernel(x, edge_index, W_l1, b_l1, W_r1, W_l2, b_l2, W_r2,
           ln1_g, ln1_b, ln2_g, ln2_b, W_c1, b_c1, W_c2, b_c2):
    src = edge_index[0]
    dst = edge_index[1]
    pad_e = E_PAD - N_EDGES
    src_p = jnp.concatenate([src, jnp.zeros((pad_e,), jnp.int32)])
    dst_p = jnp.concatenate(
        [dst, jnp.full((pad_e,), N_NODES, jnp.int32)])
    dst3d_1 = dst_p.reshape(NC * NS, NCH1, CHUNK)
    dst3d_2 = dst_p.reshape(NS, NCH2, CHUNK)
    x_p = jnp.pad(x, ((0, N_PAD - N_NODES), (0, 0)))

    agg1, degp = _sc_agg1(x_p, src_p, dst3d_1)

    row = lambda v: v.reshape(1, -1)
    h1, deg = _tc_layer1(
        agg1, degp.reshape(NC, N_PAD, 1), x_p, W_l1, W_r1, row(b_l1),
        row(ln1_g), row(ln1_b))

    agg2 = _sc_agg2(h1.reshape(NC * N_PAD, HID // 2), src_p, dst3d_2)

    wc2_p = jnp.pad(W_c2, ((0, 0), (0, HID // 2 - W_c2.shape[1])))
    bc2_p = row(jnp.pad(b_c2, (0, HID // 2 - b_c2.shape[0])))
    logits_p = _tc_layer2(h1, agg2, deg, W_l2, W_r2, row(b_l2),
                          row(ln2_g), row(ln2_b), W_c1, row(b_c1),
                          wc2_p, bc2_p)
    return logits_p[:N_NODES, :W_c2.shape[1]]


# speedup vs baseline: 9.9231x; 2.8806x over previous
import functools

import jax
import jax.numpy as jnp
from jax import lax
from jax.experimental import pallas as pl
from jax.experimental.pallas import tpu as pltpu
from jax.experimental.pallas import tpu_sc as plsc

N_NODES = 10000
N_EDGES = 320000
F_IN = 128
HID = 256

NC = 2
NS = 16
CHUNK = 128

N_PAD = 10112
E_PAD = 327680
G = 8
EV_T1 = E_PAD // (NC * NS)
EV_T2 = E_PAD // NS
NCH1 = EV_T1 // CHUNK
NCH2 = EV_T2 // CHUNK
ROWS_T = N_PAD // NS


def _fill_zeros(ref, rows):
    z = jnp.zeros((16,), jnp.float32)
    for i in range(rows):
        for j in range(ref.shape[-1] // 16):
            ref[i, pl.ds(j * 16, 16)] = z


def _edge_pipeline(nch, wid, src_hbm, dst_hbm, ev_base, table_hbm, src_b,
                   dst_b, rows, acc_sh, deg_sh, ones, ssem, dsem, rsem,
                   wsem, gsem, idx_off):
    GC = G * CHUNK
    ngroups = nch // G

    def idx_load(g, b):
        s_d = pltpu.async_copy(src_hbm.at[pl.ds(ev_base + g * GC, GC)],
                               src_b.at[b], ssem[b])
        d_d = pltpu.async_copy(dst_hbm.at[wid, pl.ds(g * G, G)],
                               dst_b.at[b], dsem[b])
        return s_d, d_d

    def wait_idx(b):
        pltpu.make_async_copy(src_hbm.at[pl.ds(0, GC)], src_b.at[b],
                              ssem[b]).wait()
        pltpu.make_async_copy(dst_hbm.at[0, pl.ds(0, G)], dst_b.at[b],
                              dsem[b]).wait()

    def offset_idx(b):
        if idx_off is not None:
            off = jnp.zeros((16,), jnp.int32) + idx_off
            for k in range(GC // 16):
                src_b[b, pl.ds(k * 16, 16)] = src_b[b, pl.ds(k * 16, 16)] + off

    def issue_gather(idx_slice, p):
        pltpu.async_copy(table_hbm.at[idx_slice], rows.at[p], rsem[p])

    def wait_gather(p):
        pltpu.make_async_copy(table_hbm.at[src_b.at[0, pl.ds(0, CHUNK)]],
                              rows.at[p], rsem[p]).wait()

    def issue_scatter(half, j, p):
        pltpu.async_copy(rows.at[p], acc_sh.at[dst_b.at[half, j]], wsem[p],
                         add=True)
        if deg_sh is not None:
            pltpu.async_copy(ones, deg_sh.at[dst_b.at[half, j]], gsem,
                             add=True)

    def wait_scatter(q):
        pltpu.make_async_copy(rows.at[q], acc_sh.at[dst_b.at[0, 0]],
                              wsem[q]).wait()

    def wait_deg():
        if deg_sh is not None:
            pltpu.make_async_copy(ones, deg_sh.at[dst_b.at[0, 0]],
                                  gsem).wait()

    s_d, d_d = idx_load(0, 0)
    s_d.wait()
    d_d.wait()
    offset_idx(0)
    issue_gather(src_b.at[0, pl.ds(0, CHUNK)], 0)

    def body(bi, carry):
        for half in range(2):
            gbase = bi * 2 + half

            for j in range(G):
                p = j % 2
                ch = gbase * G + j
                wait_gather(p)
                issue_scatter(half, j, p)

                @pl.when(ch >= 1)
                def _():
                    wait_scatter(1 - p)
                    wait_deg()

                if j == 0:
                    @pl.when(gbase + 1 < ngroups)
                    def _():
                        idx_load(gbase + 1, 1 - half)

                if j < G - 1:
                    nslice = src_b.at[half, pl.ds((j + 1) * CHUNK, CHUNK)]

                    @pl.when(ch + 1 < nch)
                    def _():
                        issue_gather(nslice, 1 - p)
                else:
                    nslice = src_b.at[1 - half, pl.ds(0, CHUNK)]

                    @pl.when(ch + 1 < nch)
                    def _():
                        wait_idx(1 - half)
                        offset_idx(1 - half)
                        issue_gather(nslice, 1 - p)

        return carry

    lax.fori_loop(0, ngroups // 2, body, 0)

    wait_scatter(1)
    wait_deg()


def _zero_acc(tid, acc_sh, deg_sh, zrow, zsem):
    def zbody(k, carry):
        r0 = tid * ROWS_T + k * 8
        pltpu.async_copy(zrow, acc_sh.at[pl.ds(r0, 8)], zsem)
        if deg_sh is not None:
            pltpu.async_copy(zrow.at[0, pl.ds(0, 8)], deg_sh.at[pl.ds(r0, 8)],
                             zsem)
        return carry

    def dbody(k, carry):
        pltpu.make_async_copy(zrow, acc_sh.at[pl.ds(0, 8)], zsem).wait()
        if deg_sh is not None:
            pltpu.make_async_copy(zrow.at[0, pl.ds(0, 8)],
                                  deg_sh.at[pl.ds(0, 8)], zsem).wait()
        return carry

    lax.fori_loop(0, ROWS_T // 8, zbody, 0)
    lax.fori_loop(0, ROWS_T // 8, dbody, 0)


def _writeback(tid, acc_sh, out_hbm, core, rows, wsem):
    r0 = tid * ROWS_T
    for k in range(4):
        p = k % 2
        if k >= 2:
            pltpu.make_async_copy(rows.at[p],
                                  out_hbm.at[core, pl.ds(r0, 128)],
                                  wsem[p]).wait()
        pltpu.sync_copy(acc_sh.at[pl.ds(r0 + k * 128, 128)], rows.at[p])
        pltpu.async_copy(rows.at[p], out_hbm.at[core, pl.ds(r0 + k * 128, 128)],
                         wsem[p])
    tail = rows.at[0].at[pl.ds(0, ROWS_T - 512)]
    pltpu.make_async_copy(rows.at[0], out_hbm.at[core, pl.ds(r0, 128)],
                          wsem[0]).wait()
    pltpu.sync_copy(acc_sh.at[pl.ds(r0 + 512, ROWS_T - 512)], tail)
    pltpu.async_copy(tail, out_hbm.at[core, pl.ds(r0 + 512, ROWS_T - 512)],
                     wsem[0])
    pltpu.make_async_copy(tail, out_hbm.at[core, pl.ds(r0, ROWS_T - 512)],
                          wsem[0]).wait()
    pltpu.make_async_copy(rows.at[1], out_hbm.at[core, pl.ds(r0, 128)],
                          wsem[1]).wait()


@functools.partial(
    pl.kernel,
    mesh=plsc.VectorSubcoreMesh(core_axis_name="c", subcore_axis_name="s"),
    out_type=(
        jax.ShapeDtypeStruct((NC, N_PAD, F_IN), jnp.float32),
        jax.ShapeDtypeStruct((NC * N_PAD,), jnp.float32),
    ),
    scratch_types=(
        pltpu.VMEM_SHARED((N_PAD, F_IN), jnp.float32),
        pltpu.VMEM_SHARED((N_PAD,), jnp.float32),
        pltpu.VMEM((2, G * CHUNK), jnp.int32),
        pltpu.VMEM((2, G, CHUNK), jnp.int32),
        pltpu.VMEM((2, CHUNK, F_IN), jnp.float32),
        pltpu.VMEM((8, F_IN), jnp.float32),
        pltpu.VMEM((CHUNK,), jnp.float32),
        pltpu.VMEM((ROWS_T,), jnp.float32),
    ) + (pltpu.SemaphoreType.DMA,) * 9,
)
def _sc_agg1(x_hbm, src_hbm, dst_hbm, agg_out, deg_out,
             acc_sh, deg_sh, src_b, dst_b, rows, zrow, ones, degb,
             ssem0, ssem1, dsem0, dsem1, rsem0, rsem1, wsem0, wsem1, gsem):
    c = lax.axis_index("c")
    s = lax.axis_index("s")
    tid = s
    wid = c * NS + s

    _fill_zeros(zrow, 8)
    o = jnp.full((16,), 1.0, jnp.float32)
    for j in range(CHUNK // 16):
        ones[pl.ds(j * 16, 16)] = o

    _zero_acc(tid, acc_sh, deg_sh, zrow, gsem)
    plsc.subcore_barrier()

    _edge_pipeline(NCH1, wid, src_hbm, dst_hbm, wid * EV_T1, x_hbm, src_b,
                   dst_b, rows, acc_sh, deg_sh, ones, (ssem0, ssem1),
                   (dsem0, dsem1), (rsem0, rsem1), (wsem0, wsem1), gsem,
                   None)
    plsc.subcore_barrier()

    _writeback(tid, acc_sh, agg_out, c, rows, (wsem0, wsem1))
    pltpu.sync_copy(deg_sh.at[pl.ds(tid * ROWS_T, ROWS_T)], degb)
    pltpu.sync_copy(degb, deg_out.at[pl.ds(c * N_PAD + tid * ROWS_T, ROWS_T)])


@functools.partial(
    pl.kernel,
    mesh=plsc.VectorSubcoreMesh(core_axis_name="c", subcore_axis_name="s"),
    out_type=jax.ShapeDtypeStruct((NC, N_PAD, HID // 2), jnp.float32),
    scratch_types=(
        pltpu.VMEM_SHARED((N_PAD, HID // 2), jnp.float32),
        pltpu.VMEM((2, G * CHUNK), jnp.int32),
        pltpu.VMEM((2, G, CHUNK), jnp.int32),
        pltpu.VMEM((2, CHUNK, HID // 2), jnp.float32),
        pltpu.VMEM((8, HID // 2), jnp.float32),
    ) + (pltpu.SemaphoreType.DMA,) * 8,
)
def _sc_agg2(h1_hbm, src_hbm, dst_hbm, agg_out,
             acc_sh, src_b, dst_b, rows, zrow,
             ssem0, ssem1, dsem0, dsem1, rsem0, rsem1, wsem0, wsem1):
    c = lax.axis_index("c")
    s = lax.axis_index("s")
    tid = s

    _fill_zeros(zrow, 8)
    _zero_acc(tid, acc_sh, None, zrow, wsem0)
    plsc.subcore_barrier()

    _edge_pipeline(NCH2, s, src_hbm, dst_hbm, s * EV_T2, h1_hbm, src_b,
                   dst_b, rows, acc_sh, None, None, (ssem0, ssem1),
                   (dsem0, dsem1), (rsem0, rsem1), (wsem0, wsem1), None,
                   c * N_NODES)
    plsc.subcore_barrier()

    _writeback(tid, acc_sh, agg_out, c, rows, (wsem0, wsem1))


_BLK = 2000
_GRID = N_NODES // _BLK
NCLS = 40


def _tc1_body(agg_ref, degp_ref, x_ref, wl_ref, wr_ref, b_ref, g_ref,
              bb_ref, h1_ref, deg_ref):
    deg = jnp.maximum(degp_ref[0] + degp_ref[1], 1.0)
    mean = (agg_ref[0] + agg_ref[1]) / deg
    h = (jnp.dot(mean, wl_ref[...], preferred_element_type=jnp.float32)
         + jnp.dot(x_ref[...], wr_ref[...], preferred_element_type=jnp.float32)
         + b_ref[...])
    mu = jnp.mean(h, axis=1, keepdims=True)
    var = jnp.mean((h - mu) * (h - mu), axis=1, keepdims=True)
    h = (h - mu) * lax.rsqrt(var + 1e-5) * g_ref[...] + bb_ref[...]
    h = jnp.maximum(h, 0.0)
    h1_ref[0] = h[:, :HID // 2]
    h1_ref[1] = h[:, HID // 2:]
    deg_ref[...] = deg


def _tc_layer1(agg, degp, x, wl, wr, b, g, bb):
    full = lambda s: pl.BlockSpec(s, lambda i: (0,) * len(s))
    return pl.pallas_call(
        _tc1_body,
        grid=(_GRID,),
        in_specs=[
            pl.BlockSpec((NC, _BLK, F_IN), lambda i: (0, i, 0)),
            pl.BlockSpec((NC, _BLK, 1), lambda i: (0, i, 0)),
            pl.BlockSpec((_BLK, F_IN), lambda i: (i, 0)),
            full((F_IN, HID)), full((F_IN, HID)), full((1, HID)),
            full((1, HID)), full((1, HID)),
        ],
        out_specs=[
            pl.BlockSpec((NC, _BLK, HID // 2), lambda i: (0, i, 0)),
            pl.BlockSpec((_BLK, 1), lambda i: (i, 0)),
        ],
        out_shape=[
            jax.ShapeDtypeStruct((NC, N_NODES, HID // 2), jnp.float32),
            jax.ShapeDtypeStruct((N_NODES, 1), jnp.float32),
        ],
    )(agg, degp, x, wl, wr, b, g, bb)


def _tc2_body(h1_ref, agg_ref, deg_ref, wl_ref, wr_ref, b_ref,
              g_ref, bb_ref, wc1_ref, bc1_ref, wc2_ref, bc2_ref, out_ref):
    rdeg = 1.0 / deg_ref[...]
    wl = wl_ref[...]
    wr = wr_ref[...]
    dot = lambda a, w: jnp.dot(a, w, preferred_element_type=jnp.float32)
    h = (dot(agg_ref[0] * rdeg, wl[:HID // 2])
         + dot(agg_ref[1] * rdeg, wl[HID // 2:])
         + dot(h1_ref[0], wr[:HID // 2])
         + dot(h1_ref[1], wr[HID // 2:])
         + b_ref[...])
    mu = jnp.mean(h, axis=1, keepdims=True)
    var = jnp.mean((h - mu) * (h - mu), axis=1, keepdims=True)
    h = (h - mu) * lax.rsqrt(var + 1e-5) * g_ref[...] + bb_ref[...]
    h = jnp.maximum(h, 0.0)
    z = jnp.maximum(dot(h, wc1_ref[...]) + bc1_ref[...], 0.0)
    out_ref[...] = dot(z, wc2_ref[...]) + bc2_ref[...]


def _tc_layer2(h1, agg, deg, wl, wr, b, g, bb, wc1, bc1, wc2, bc2):
    full = lambda s: pl.BlockSpec(s, lambda i: (0,) * len(s))
    return pl.pallas_call(
        _tc2_body,
        grid=(_GRID,),
        in_specs=[
            pl.BlockSpec((NC, _BLK, HID // 2), lambda i: (0, i, 0)),
            pl.BlockSpec((NC, _BLK, HID // 2), lambda i: (0, i, 0)),
            pl.BlockSpec((_BLK, 1), lambda i: (i, 0)),
            full((HID, HID)), full((HID, HID)), full((1, HID)),
            full((1, HID)), full((1, HID)),
            full((HID, HID // 2)), full((1, HID // 2)),
            full((HID // 2, NCLS)), full((1, NCLS)),
        ],
        out_specs=pl.BlockSpec((_BLK, NCLS), lambda i: (i, 0)),
        out_shape=jax.ShapeDtypeStruct((N_NODES, NCLS), jnp.float32),
    )(h1, agg, deg, wl, wr, b, g, bb, wc1, bc1, wc2, bc2)


def kernel(x, edge_index, W_l1, b_l1, W_r1, W_l2, b_l2, W_r2,
           ln1_g, ln1_b, ln2_g, ln2_b, W_c1, b_c1, W_c2, b_c2):
    src = edge_index[0]
    dst = edge_index[1]
    pad_e = E_PAD - N_EDGES
    pad_i = jnp.arange(pad_e, dtype=jnp.int32)
    src_p = jnp.concatenate([src, pad_i % N_NODES])
    dst_p = jnp.concatenate([dst, N_NODES + pad_i % (N_PAD - N_NODES)])
    dst3d_1 = dst_p.reshape(NC * NS, NCH1, CHUNK)
    dst3d_2 = dst_p.reshape(NS, NCH2, CHUNK)

    agg1, degp = _sc_agg1(x, src_p, dst3d_1)

    row = lambda v: v.reshape(1, -1)
    h1, deg = _tc_layer1(
        agg1, degp.reshape(NC, N_PAD, 1), x, W_l1, W_r1, row(b_l1),
        row(ln1_g), row(ln1_b))

    agg2 = _sc_agg2(h1.reshape(NC * N_NODES, HID // 2), src_p, dst3d_2)

    return _tc_layer2(h1, agg2, deg, W_l2, W_r2, row(b_l2),
                      row(ln2_g), row(ln2_b), W_c1, row(b_c1),
                      W_c2, row(b_c2))
